# baseline (speedup 1.0000x reference)
import functools

import jax
import jax.numpy as jnp
from jax import lax
from jax.experimental import pallas as pl
from jax.experimental.pallas import tpu as pltpu
from jax.experimental.pallas import tpu_sc as plsc

_F = 26
_V = 100000
_D = 32
_B = 16384
_NW = 32
_BH = _B // 2

_mesh = plsc.VectorSubcoreMesh(core_axis_name="c", subcore_axis_name="s")


def _make_group_kernel(f0, nf):
    ncols = nf * _D

    @functools.partial(
        pl.kernel,
        mesh=_mesh,
        out_type=jax.ShapeDtypeStruct((ncols, _B), jnp.float32),
        compiler_params=pltpu.CompilerParams(
            use_tc_tiling_on_sc=False, needs_layout_passes=False),
        scratch_types=[
            pltpu.VMEM((_V,), jnp.float32),
            pltpu.VMEM((_BH,), jnp.int32),
            pltpu.VMEM((_BH,), jnp.float32),
            pltpu.SemaphoreType.DMA,
        ],
    )
    def group_kernel(xt_hbm, tab_hbm, out_hbm, slicev, xfv, colv, sem):
        wid = lax.axis_index("s") * 2 + lax.axis_index("c")

        def task_body(t, carry):
            c = t * _NW + wid
            f = c // _D
            d = c - f * _D
            pltpu.sync_copy(tab_hbm.at[f, d], slicev)

            def half_body(h, carry2):
                pltpu.sync_copy(xt_hbm.at[f0 + f, pl.ds(h * _BH, _BH)], xfv)

                def vec_body(k, carry3):
                    for u in range(8):
                        sl = pl.ds(k * 128 + u * 16, 16)
                        colv[sl] = plsc.load_gather(slicev, [xfv[sl]])
                    return carry3

                lax.fori_loop(0, _BH // 128, vec_body, 0)
                pltpu.sync_copy(colv, out_hbm.at[c, pl.ds(h * _BH, _BH)])
                return carry2

            lax.fori_loop(0, 2, half_body, 0)
            return carry

        lax.fori_loop(0, ncols // _NW, task_body, 0)

    return group_kernel


_GROUPS = ((0, 7), (7, 7), (14, 6), (20, 6))
_KERNELS = tuple(_make_group_kernel(f0, nf) for f0, nf in _GROUPS)


@jax.jit
def kernel(x, tables):
    xt = jnp.transpose(x)
    outs = [
        k(xt, jnp.transpose(tables[f0:f0 + nf], (0, 2, 1)))
        for k, (f0, nf) in zip(_KERNELS, _GROUPS)
    ]
    out_t = jnp.concatenate(outs, axis=0)
    return jnp.transpose(out_t)


# speedup vs baseline: 1.0230x; 1.0230x over previous
import functools

import jax
import jax.numpy as jnp
from jax import lax
from jax.experimental import pallas as pl
from jax.experimental.pallas import tpu as pltpu
from jax.experimental.pallas import tpu_sc as plsc

_F = 26
_V = 100000
_VH = _V // 2
_D = 32
_B = 16384
_NW = 32
_BH = _B // 2
_NT = _F * _D
_TPW = _NT // _NW

_mesh = plsc.VectorSubcoreMesh(core_axis_name="c", subcore_axis_name="s")


@functools.partial(
    pl.kernel,
    mesh=_mesh,
    out_type=jax.ShapeDtypeStruct((_NT, _B), jnp.float32),
    compiler_params=pltpu.CompilerParams(
        use_tc_tiling_on_sc=False, needs_layout_passes=False),
    scratch_types=[
        pltpu.VMEM((_VH,), jnp.float32),
        pltpu.VMEM((_VH,), jnp.float32),
        pltpu.VMEM((_B,), jnp.int32),
        pltpu.VMEM((_BH,), jnp.float32),
        pltpu.SemaphoreType.DMA,
        pltpu.SemaphoreType.DMA,
    ],
)
def _embed_cols(xt_hbm, tab_hbm, out_hbm, sloA, sloB, xfv, colv, mA, mB):
    wid = lax.axis_index("s") * 2 + lax.axis_index("c")

    def fire_a(t):
        c = t * _NW + wid
        f = c // _D
        d = c - f * _D
        return pltpu.async_copy(tab_hbm.at[f, d, pl.ds(0, _VH)], sloA, mA)

    def pass_lo(h):
        def body(k, carry):
            for u in range(8):
                sl = pl.ds(h * _BH + k * 128 + u * 16, 16)
                osl = pl.ds(k * 128 + u * 16, 16)
                v = xfv[sl]
                m = v < _VH
                g = plsc.load_gather(sloA, [jnp.where(m, v, 0)])
                colv[osl] = g
            return carry
        lax.fori_loop(0, _BH // 128, body, 0)

    def pass_hi(h):
        def body(k, carry):
            for u in range(8):
                sl = pl.ds(h * _BH + k * 128 + u * 16, 16)
                osl = pl.ds(k * 128 + u * 16, 16)
                v = xfv[sl]
                m = v >= _VH
                g = plsc.load_gather(sloB, [jnp.where(m, v - _VH, 0)])
                colv[osl] = jnp.where(m, g, colv[osl])
            return carry
        lax.fori_loop(0, _BH // 128, body, 0)

    fire_a(0)

    def task_body(t, carry):
        c = t * _NW + wid
        f = c // _D
        d = c - f * _D
        pltpu.sync_copy(xt_hbm.at[f], xfv)
        pltpu.make_async_copy(tab_hbm.at[f, d, pl.ds(0, _VH)], sloA, mA).wait()
        cpB = pltpu.async_copy(tab_hbm.at[f, d, pl.ds(_VH, _VH)], sloB, mB)
        pass_lo(0)
        cpB.wait()
        pass_hi(0)
        pltpu.sync_copy(colv, out_hbm.at[c, pl.ds(0, _BH)])
        pass_lo(1)

        @pl.when(t + 1 < _TPW)
        def _():
            nc = (t + 1) * _NW + wid
            nf = nc // _D
            nd = nc - nf * _D
            pltpu.async_copy(tab_hbm.at[nf, nd, pl.ds(0, _VH)], sloA, mA)

        pass_hi(1)
        pltpu.sync_copy(colv, out_hbm.at[c, pl.ds(_BH, _BH)])
        return carry

    lax.fori_loop(0, _TPW, task_body, 0)


@jax.jit
def kernel(x, tables):
    xt = jnp.transpose(x)
    tt = jnp.transpose(tables, (0, 2, 1))
    out_t = _embed_cols(xt, tt)
    return jnp.transpose(out_t)


# speedup vs baseline: 1.0529x; 1.0292x over previous
import functools

import jax
import jax.numpy as jnp
from jax import lax
from jax.experimental import pallas as pl
from jax.experimental.pallas import tpu as pltpu
from jax.experimental.pallas import tpu_sc as plsc

_F = 26
_V = 100000
_D = 32
_B = 16384
_NW = 32
_BH = _B // 2

_mesh = plsc.VectorSubcoreMesh(core_axis_name="c", subcore_axis_name="s")


def _make_group_kernel(f0, nf):
    ncols = nf * _D

    @functools.partial(
        pl.kernel,
        mesh=_mesh,
        out_type=jax.ShapeDtypeStruct((ncols, _B), jnp.float32),
        compiler_params=pltpu.CompilerParams(
            use_tc_tiling_on_sc=False, needs_layout_passes=False),
        scratch_types=[
            pltpu.VMEM((_V,), jnp.float32),
            pltpu.VMEM((_BH,), jnp.int32),
            pltpu.VMEM((_BH,), jnp.float32),
            pltpu.SemaphoreType.DMA,
        ],
    )
    def group_kernel(xt_hbm, tab_hbm, out_hbm, slicev, xfv, colv, sem):
        wid = lax.axis_index("s") * 2 + lax.axis_index("c")

        def task_body(t, carry):
            c = t * _NW + wid
            f = c // _D
            d = c - f * _D
            pltpu.sync_copy(tab_hbm.at[f0 + f, d], slicev)

            def half_body(h, carry2):
                pltpu.sync_copy(xt_hbm.at[f0 + f, pl.ds(h * _BH, _BH)], xfv)

                def vec_body(k, carry3):
                    for u in range(8):
                        sl = pl.ds(k * 128 + u * 16, 16)
                        colv[sl] = plsc.load_gather(slicev, [xfv[sl]])
                    return carry3

                lax.fori_loop(0, _BH // 128, vec_body, 0)
                pltpu.sync_copy(colv, out_hbm.at[c, pl.ds(h * _BH, _BH)])
                return carry2

            lax.fori_loop(0, 2, half_body, 0)
            return carry

        lax.fori_loop(0, ncols // _NW, task_body, 0)

    return group_kernel


_GROUPS = ((0, 13), (13, 13))
_KERNELS = tuple(_make_group_kernel(f0, nf) for f0, nf in _GROUPS)


@jax.jit
def kernel(x, tables):
    xt = jnp.transpose(x)
    tt = jnp.transpose(tables, (0, 2, 1))
    outs = [k(xt, tt) for k in _KERNELS]
    out_t = jnp.concatenate(outs, axis=0)
    return jnp.transpose(out_t)


# speedup vs baseline: 1.0803x; 1.0260x over previous
import functools

import jax
import jax.numpy as jnp
from jax import lax
from jax.experimental import pallas as pl
from jax.experimental.pallas import tpu as pltpu
from jax.experimental.pallas import tpu_sc as plsc

_F = 26
_V = 100000
_D = 32
_B = 16384
_NW = 32
_NT = _F * _D
_BH = _B // 2

_mesh = plsc.VectorSubcoreMesh(core_axis_name="c", subcore_axis_name="s")


@functools.partial(
    pl.kernel,
    mesh=_mesh,
    out_type=jax.ShapeDtypeStruct((_NT, _B), jnp.float32),
    compiler_params=pltpu.CompilerParams(
        use_tc_tiling_on_sc=False, needs_layout_passes=False),
    scratch_types=[
        pltpu.VMEM((_V,), jnp.float32),
        pltpu.VMEM((_BH,), jnp.int32),
        pltpu.VMEM((_BH,), jnp.float32),
        pltpu.SemaphoreType.DMA,
    ],
)
def _embed_cols(xt_hbm, tab_hbm, out_hbm, slicev, xfv, colv, sem):
    wid = lax.axis_index("s") * 2 + lax.axis_index("c")

    def task_body(t, carry):
        c = t * _NW + wid
        f = c // _D
        d = c - f * _D
        pltpu.sync_copy(tab_hbm.at[f, d], slicev)

        def half_body(h, carry2):
            pltpu.sync_copy(xt_hbm.at[f, pl.ds(h * _BH, _BH)], xfv)

            def vec_body(k, carry3):
                for u in range(8):
                    sl = pl.ds(k * 128 + u * 16, 16)
                    colv[sl] = plsc.load_gather(slicev, [xfv[sl]])
                return carry3

            lax.fori_loop(0, _BH // 128, vec_body, 0)
            pltpu.sync_copy(colv, out_hbm.at[c, pl.ds(h * _BH, _BH)])
            return carry2

        lax.fori_loop(0, 2, half_body, 0)
        return carry

    lax.fori_loop(0, _NT // _NW, task_body, 0)


@jax.jit
def kernel(x, tables):
    xt = jnp.transpose(x)
    tt = jnp.transpose(tables, (0, 2, 1))
    out_t = _embed_cols(xt, tt)
    return jnp.transpose(out_t)


# speedup vs baseline: 1.0969x; 1.0154x over previous
import functools

import jax
import jax.numpy as jnp
from jax import lax
from jax.experimental import pallas as pl
from jax.experimental.pallas import tpu as pltpu
from jax.experimental.pallas import tpu_sc as plsc

_F = 26
_V = 100000
_D = 32
_B = 16384
_NW = 32
_NT = _F * _D
_BH = _B // 2

_mesh = plsc.VectorSubcoreMesh(core_axis_name="c", subcore_axis_name="s")


@functools.partial(
    pl.kernel,
    mesh=_mesh,
    out_type=jax.ShapeDtypeStruct((_NT, _B), jnp.float32),
    compiler_params=pltpu.CompilerParams(
        use_tc_tiling_on_sc=False, needs_layout_passes=False),
    scratch_types=[
        pltpu.VMEM((_V,), jnp.float32),
        pltpu.VMEM((_B,), jnp.int32),
        pltpu.VMEM((_BH,), jnp.float32),
        pltpu.SemaphoreType.DMA,
    ],
)
def _embed_cols(xt_hbm, tab_hbm, out_hbm, slicev, xfv, colv, sem):
    wid = lax.axis_index("s") * 2 + lax.axis_index("c")

    def task_body(t, carry):
        c = t * _NW + wid
        f = c // _D
        d = c - f * _D
        cps = pltpu.async_copy(tab_hbm.at[f, d], slicev, sem)
        pltpu.sync_copy(xt_hbm.at[f], xfv)
        cps.wait()

        def half_body(h, carry2):
            def vec_body(k, carry3):
                for u in range(8):
                    sl = pl.ds(h * _BH + k * 128 + u * 16, 16)
                    osl = pl.ds(k * 128 + u * 16, 16)
                    colv[osl] = plsc.load_gather(slicev, [xfv[sl]])
                return carry3

            lax.fori_loop(0, _BH // 128, vec_body, 0)
            pltpu.sync_copy(colv, out_hbm.at[c, pl.ds(h * _BH, _BH)])
            return carry2

        lax.fori_loop(0, 2, half_body, 0)
        return carry

    lax.fori_loop(0, _NT // _NW, task_body, 0)


@jax.jit
def kernel(x, tables):
    xt = jnp.transpose(x)
    tt = jnp.transpose(tables, (0, 2, 1))
    out_t = _embed_cols(xt, tt)
    return jnp.transpose(out_t)
